# baseline (speedup 1.0000x reference)
import functools

import jax
import jax.numpy as jnp
from jax import lax
from jax.experimental import pallas as pl
from jax.experimental.pallas import tpu as pltpu
from jax.experimental.pallas import tpu_sc as plsc

VOCAB = 100000
EMBED_DIM = 128
B_TOTAL = 4096 * 200

NC = 2
NS = 16
NW = NC * NS
B_PER_W = B_TOTAL // NW
CHUNK = 128
PAIR = 2 * CHUNK
N_PAIRS = B_PER_W // PAIR
S = 2


def _make_gather():
    mesh = plsc.VectorSubcoreMesh(core_axis_name="c", subcore_axis_name="s")

    @functools.partial(
        pl.kernel,
        mesh=mesh,
        out_type=jax.ShapeDtypeStruct((B_TOTAL, EMBED_DIM), jnp.float32),
        scratch_types=[
            pltpu.VMEM((B_PER_W,), jnp.int32),
        ]
        + [pltpu.VMEM((PAIR, EMBED_DIM), jnp.float32) for _ in range(S)]
        + [pltpu.SemaphoreType.DMA for _ in range(2 * S)],
    )
    def k(table_hbm, idx_hbm, out_hbm, idx_v, *bufs_and_sems):
        rows = bufs_and_sems[:S]
        gsem = bufs_and_sems[S:2 * S]
        wsem = bufs_and_sems[2 * S:]
        wid = lax.axis_index("s") * NC + lax.axis_index("c")
        base = wid * B_PER_W

        pltpu.sync_copy(idx_hbm.at[pl.ds(base, B_PER_W)], idx_v)

        def g_descs(j, s):
            return [
                pltpu.make_async_copy(
                    table_hbm.at[idx_v.at[pl.ds(j * PAIR + h * CHUNK, CHUNK)]],
                    rows[s].at[pl.ds(h * CHUNK, CHUNK), :],
                    gsem[s],
                )
                for h in range(2)
            ]

        def w_desc(j, s):
            return pltpu.make_async_copy(
                rows[s], out_hbm.at[pl.ds(base + j * PAIR, PAIR)], wsem[s]
            )

        def step(i, s, first, prefetch):
            for d in g_descs(i, s):
                d.wait()
            w_desc(i, s).start()
            w_desc(i, s).wait()
            if prefetch:
                for d in g_descs(i + S, s):
                    d.start()

        for j in range(S):
            for d in g_descs(j, j):
                d.start()
        for i in range(S):
            step(i, i, True, True)

        def body(g, carry):
            i0 = g * S
            for b in range(S):
                step(i0 + b, b, False, True)
            return carry

        lax.fori_loop(1, N_PAIRS // S - 1, body, 0)

        for b in range(S):
            i = N_PAIRS - S + b
            step(i, b, False, False)

    return k


_gather = _make_gather()


def kernel(doc, W):
    idx = doc.reshape(-1).astype(jnp.int32)
    out = _gather(W, idx)
    return out.reshape(doc.shape[0], doc.shape[1], EMBED_DIM)
